# Initial kernel scaffold; baseline (speedup 1.0000x reference)
import jax
import jax.numpy as jnp
from jax.experimental import pallas as pl


def kernel(step_in_seq, lprobs, scores):
    raise NotImplementedError("write your pallas kernel here")


# speedup vs baseline: 2.8648x; 2.8648x over previous
import functools

import jax
import jax.numpy as jnp
from jax import lax
from jax.experimental import pallas as pl
from jax.experimental.pallas import tpu as pltpu
from jax.experimental.pallas import tpu_sc as plsc

BSZ = 64
BEAMS = 4
VOCAB = 100000
FLAT = BEAMS * VOCAB
K_OUT = 8

LANES = 16
CHUNK = 20000
NCHUNKS = VOCAB // CHUNK
GROUP_VREGS = 25
GROUP = GROUP_VREGS * LANES
NGROUPS = CHUNK // GROUP

NWORK = 32
ROWS_PER_W = BSZ // NWORK

NEG_INF = float("-inf")


def _merge_topk(T, TI, v, vi):
    vd, vdi = plsc.sort_key_val(v, vi, descending=True)
    keep = (T > vd) | ((T == vd) & (TI < vdi))
    newT = jnp.maximum(T, vd)
    newTI = jnp.where(keep, TI, vdi)
    sT, sTI = plsc.sort_key_val(newT, newTI, descending=False)
    return sT, sTI


def _tile_body(lprobs_hbm, bias_hbm, vals_hbm, toks_hbm, beams_hbm,
               buf, bias_v, out_v, out_ti, out_bi, sem):
    cid = lax.axis_index("c")
    sid = lax.axis_index("s")
    wid = sid * 2 + cid

    iota = lax.iota(jnp.int32, LANES)

    for r in range(ROWS_PER_W):
        row = wid * ROWS_PER_W + r
        pltpu.sync_copy(bias_hbm.at[row], bias_v)

        T = jnp.full((LANES,), NEG_INF, jnp.float32)
        TI = jnp.zeros((LANES,), jnp.int32)

        for b in range(BEAMS):
            biasv = bias_v[b]
            bias_s = lax.reduce_max(biasv, axes=(0,))
            thr_raw = lax.reduce_min(T, axes=(0,)) - bias_s

            def chunk_body(c, carry, b=b, biasv=biasv, bias_s=bias_s, row=row):
                T, TI, thr_raw = carry
                off = b * VOCAB + c * CHUNK
                pltpu.sync_copy(lprobs_hbm.at[row, pl.ds(off, CHUNK)], buf)

                def group_body(g, carry2, off=off, biasv=biasv, bias_s=bias_s):
                    T, TI, thr_raw = carry2
                    base = g * GROUP
                    gmax = buf[pl.ds(base, LANES)]
                    for j in range(1, GROUP_VREGS):
                        gmax = jnp.maximum(gmax, buf[pl.ds(base + j * LANES, LANES)])
                    gmax_s = lax.reduce_max(gmax, axes=(0,))

                    def insert(_):
                        def ins_j(j, carry3):
                            T, TI = carry3
                            p = base + j * LANES
                            v = buf[pl.ds(p, LANES)] + biasv
                            vi = iota + (off + p)
                            return _merge_topk(T, TI, v, vi)
                        T2, TI2 = lax.fori_loop(0, GROUP_VREGS, ins_j, (T, TI))
                        thr2 = lax.reduce_min(T2, axes=(0,)) - bias_s
                        return T2, TI2, thr2

                    def skip(_):
                        return T, TI, thr_raw

                    return lax.cond(gmax_s > thr_raw, insert, skip, None)

                return lax.fori_loop(0, NGROUPS, group_body, (T, TI, thr_raw))

            T, TI, thr_raw = lax.fori_loop(0, NCHUNKS, chunk_body,
                                           (T, TI, thr_raw))

        Td = lax.rev(T, dimensions=(0,))
        TId = lax.rev(TI, dimensions=(0,))
        out_v[...] = Td
        out_ti[...] = TId % VOCAB
        out_bi[...] = TId // VOCAB
        pltpu.sync_copy(out_v, vals_hbm.at[row])
        pltpu.sync_copy(out_ti, toks_hbm.at[row])
        pltpu.sync_copy(out_bi, beams_hbm.at[row])


@jax.jit
def _topk_sc(lprobs_flat, bias_splat):
    kern = pl.kernel(
        _tile_body,
        out_type=(
            jax.ShapeDtypeStruct((BSZ, LANES), jnp.float32),
            jax.ShapeDtypeStruct((BSZ, LANES), jnp.int32),
            jax.ShapeDtypeStruct((BSZ, LANES), jnp.int32),
        ),
        mesh=plsc.VectorSubcoreMesh(core_axis_name="c", subcore_axis_name="s"),
        scratch_types=[
            pltpu.VMEM((CHUNK,), jnp.float32),
            pltpu.VMEM((BEAMS, LANES), jnp.float32),
            pltpu.VMEM((LANES,), jnp.float32),
            pltpu.VMEM((LANES,), jnp.int32),
            pltpu.VMEM((LANES,), jnp.int32),
            pltpu.SemaphoreType.DMA,
        ],
        compiler_params=pltpu.CompilerParams(use_tc_tiling_on_sc=False,
                                             needs_layout_passes=False),
    )
    return kern(lprobs_flat, bias_splat)


def kernel(step_in_seq, lprobs, scores):
    bsz, beam_size, vocab = lprobs.shape
    bias = lax.dynamic_index_in_dim(scores, step_in_seq - 1, axis=2,
                                    keepdims=False)
    bias_splat = jnp.broadcast_to(bias[:, :, None], (bsz, beam_size, LANES))
    lprobs_flat = lprobs.reshape(bsz, beam_size * vocab)
    vals, toks, beams = _topk_sc(lprobs_flat, bias_splat)
    return (vals[:, :K_OUT], toks[:, :K_OUT], beams[:, :K_OUT])
